# baseline (speedup 1.0000x reference)
import functools

import jax
import jax.numpy as jnp
from jax import lax
from jax.experimental import pallas as pl
from jax.experimental.pallas import tpu as pltpu
from jax.experimental.pallas import tpu_sc as plsc

B = 16384
D = 32
NC = 2
NS = 16
NW = NC * NS
BPW = B // NW
CHUNK = 128
NCH = BPW // CHUNK
L = 16
NG = BPW // L

_mesh = plsc.VectorSubcoreMesh(core_axis_name="c", subcore_axis_name="s")


@functools.partial(
    pl.kernel,
    mesh=_mesh,
    compiler_params=pltpu.CompilerParams(
        needs_layout_passes=False, use_tc_tiling_on_sc=False),
    out_type=jax.ShapeDtypeStruct((B,), jnp.float32),
    scratch_types=[
        pltpu.VMEM((NCH, CHUNK), jnp.int32),
        pltpu.VMEM((NCH, CHUNK), jnp.int32),
        pltpu.VMEM((BPW, D), jnp.float32),
        pltpu.VMEM((BPW, D), jnp.float32),
        pltpu.VMEM((BPW, D), jnp.float32),
        pltpu.VMEM((BPW, D), jnp.float32),
        pltpu.VMEM((BPW,), jnp.float32),
        pltpu.SemaphoreType.DMA,
    ],
)
def _line_sc(ui_hbm, uj_hbm, fw_hbm, sw_hbm, cw_hbm, out_hbm,
             ui_v, uj_v, a_v, b_v, c_v, e_v, o_v, sem):
    wid = lax.axis_index("s") * NC + lax.axis_index("c")
    base = wid * BPW

    pltpu.sync_copy(ui_hbm.at[wid], ui_v)
    pltpu.sync_copy(uj_hbm.at[wid], uj_v)

    descs = []
    for t in range(NCH):
        sl = pl.ds(t * CHUNK, CHUNK)
        descs.append(pltpu.async_copy(fw_hbm.at[ui_v.at[t]], a_v.at[sl], sem))
        descs.append(pltpu.async_copy(fw_hbm.at[uj_v.at[t]], b_v.at[sl], sem))
        descs.append(pltpu.async_copy(sw_hbm.at[ui_v.at[t]], c_v.at[sl], sem))
        descs.append(pltpu.async_copy(cw_hbm.at[uj_v.at[t]], e_v.at[sl], sem))
    for dsc in descs:
        dsc.wait()


    def group(g, carry):
        rows = g * L + lax.iota(jnp.int32, L)
        acc = jnp.zeros((L,), jnp.float32)
        for d in range(D):
            col = jnp.full((L,), d, jnp.int32)
            av = plsc.load_gather(a_v, [rows, col])
            bv = plsc.load_gather(b_v, [rows, col])
            cv = plsc.load_gather(c_v, [rows, col])
            ev = plsc.load_gather(e_v, [rows, col])
            acc = acc + av * bv + cv * ev
        o_v[pl.ds(g * L, L)] = acc
        return carry

    lax.fori_loop(0, NG, group, 0)

    pltpu.sync_copy(o_v, out_hbm.at[pl.ds(base, BPW)])


def kernel(u_i, u_j, first_w, second_w, context_w):
    ui3 = u_i.astype(jnp.int32).reshape(NW, NCH, CHUNK)
    uj3 = u_j.astype(jnp.int32).reshape(NW, NCH, CHUNK)
    return _line_sc(ui3, uj3, first_w, second_w, context_w)
